# Initial kernel scaffold; baseline (speedup 1.0000x reference)
import jax
import jax.numpy as jnp
from jax.experimental import pallas as pl


def kernel(x, edge_index, Wpool1, bpool1, Wself1, bself1, Wneigh1, Wpool2, bpool2, Wself2, bself2, Wneigh2, Wp, bp):
    raise NotImplementedError("write your pallas kernel here")


# speedup vs baseline: 2.9597x; 2.9597x over previous
import functools

import jax
import jax.numpy as jnp
from jax import lax
from jax.experimental import pallas as pl
from jax.experimental.pallas import tpu as pltpu
from jax.experimental.pallas import tpu_sc as plsc

N = 10000
NPAD = 10016
E = 320000
D = 128
C = 16
EPS = 1e-12

TILES = 32
NLOC = 313
NACC = NLOC + 1
CH = 2000
NCHUNK = E // CH
G = 128
NBUF = 4
PEND = 2304
CAPT = E + NBUF * G
SCH = 2000


def _l2n(h):
    n = jnp.sqrt(jnp.sum(h * h, axis=1, keepdims=True))
    return h / jnp.maximum(n, EPS)


def _tc_a(x_ref, wp_ref, bp_ref, ws_ref, bs_ref, m_ref, xs_ref):
    xb = x_ref[:]
    dn = (((1,), (1,)), ((), ()))
    m_ref[:] = jnp.maximum(
        lax.dot_general(xb, wp_ref[:], dn, preferred_element_type=jnp.float32)
        + bp_ref[:], 0.0)
    xs_ref[:] = (
        lax.dot_general(xb, ws_ref[:], dn, preferred_element_type=jnp.float32)
        + bs_ref[:])


def _tc_b(xs_ref, hn_ref, wn_ref, wp_ref, bp_ref, ws_ref, bs_ref,
          m_ref, xs2_ref):
    dn = (((1,), (1,)), ((), ()))
    h = xs_ref[:] + lax.dot_general(hn_ref[:], wn_ref[:], dn,
                                    preferred_element_type=jnp.float32)
    h = _l2n(jnp.maximum(h, 0.0))
    m_ref[:] = jnp.maximum(
        lax.dot_general(h, wp_ref[:], dn, preferred_element_type=jnp.float32)
        + bp_ref[:], 0.0)
    xs2_ref[:] = (
        lax.dot_general(h, ws_ref[:], dn, preferred_element_type=jnp.float32)
        + bs_ref[:])


def _tc_c(xs_ref, hn_ref, wn_ref, wpu_ref, wpv_ref, bp_ref,
          su_ref, sv_ref):
    dn = (((1,), (1,)), ((), ()))
    h = xs_ref[:] + lax.dot_general(hn_ref[:], wn_ref[:], dn,
                                    preferred_element_type=jnp.float32)
    h = _l2n(h)
    su_ref[:] = (
        lax.dot_general(wpu_ref[:], h, dn, preferred_element_type=jnp.float32)
        + bp_ref[:])
    sv_ref[:] = lax.dot_general(wpv_ref[:], h, dn,
                                preferred_element_type=jnp.float32)


def _tc_t(t_ref, o_ref):
    o_ref[:] = t_ref[:].T


_MESH = plsc.VectorSubcoreMesh(core_axis_name="c", subcore_axis_name="s",
                               num_cores=2, num_subcores=16)


def _m8(v):
    return pl.multiple_of(v, 8)


def _ring_accum(m_hbm, bin_hbm, tbase, total, bw_ref, gidx_ref, dlv_ref,
                rows_ref, acc_ref, sems):
    steps = (total // G) // NBUF

    def step(s, _):
        hs = []
        for b in range(NBUF):
            goff = (s * NBUF + b) * G
            pltpu.sync_copy(bin_hbm.at[pl.ds(_m8(tbase + goff), G)],
                            bw_ref.at[pl.ds(b * G, G)])
            for k in range(G // 16):
                p = bw_ref[pl.ds(b * G + 16 * k, 16)]
                gidx_ref[pl.ds(b * G + 16 * k, 16)] = \
                    lax.shift_right_logical(p, 9)
                dlv_ref[pl.ds(b * G + 16 * k, 16)] = lax.bitwise_and(p, 511)
            hs.append(pltpu.async_copy(
                m_hbm.at[gidx_ref.at[pl.ds(b * G, G)]],
                rows_ref.at[pl.ds(b * G, G)], sems[b]))
        for b in range(NBUF):
            hs[b].wait()

            def body(q, _, b=b):
                dv = dlv_ref[pl.ds(b * G + 16 * q, 16)]
                for k in range(16):
                    base = dv[k] * D
                    j = b * G + 16 * q + k
                    for cg in range(D // 16):
                        a = acc_ref[pl.ds(base + 16 * cg, 16)]
                        r = rows_ref[j, pl.ds(16 * cg, 16)]
                        acc_ref[pl.ds(base + 16 * cg, 16)] = jnp.maximum(a, r)
                return 0

            lax.fori_loop(0, G // 16, body, 0)
        return 0

    lax.fori_loop(0, steps, step, 0)


def _zero_acc(acc_ref):
    z = jnp.zeros((16,), jnp.float32)

    def zb(i, _):
        for k in range(8):
            acc_ref[pl.ds(i * 128 + 16 * k, 16)] = z
        return 0

    lax.fori_loop(0, NACC, zb, 0)


def _write_slab(acc_ref, hn_hbm, wid):
    pltpu.sync_copy(acc_ref.at[pl.ds(0, NLOC * D)],
                    hn_hbm.at[pl.ds(_m8(wid * NLOC * D), NLOC * D)])


def _sc_d1(m_hbm, src_hbm, dst_hbm, hn_hbm, bin_hbm, cnt_hbm,
           srcv, dstv, pend, gidx, dlv, rows, acc, cntv,
           sem0, sem1, sem2, sem3):
    wid = lax.axis_index("s") * 2 + lax.axis_index("c")
    lo = wid * NLOC
    hi = lo + NLOC
    tbase = wid * CAPT
    _zero_acc(acc)
    iota = lax.iota(jnp.int32, 16)

    def chunk_body(c, carry):
        rem, woff = carry
        pltpu.sync_copy(src_hbm.at[pl.ds(_m8(c * CH), CH)], srcv)
        pltpu.sync_copy(dst_hbm.at[pl.ds(_m8(c * CH), CH)], dstv)

        def scan_body(i, cnt):
            d = dstv[pl.ds(16 * i, 16)]
            s = srcv[pl.ds(16 * i, 16)]
            msk = jnp.logical_and(d >= lo, d < hi)
            p = lax.bitwise_or(lax.shift_left(s, 9),
                               lax.bitwise_and(d - lo, 511))
            cs = plsc.cumsum(msk.astype(jnp.int32))
            idx = cnt + cs - 1
            plsc.store_scatter(pend, [idx], p, mask=msk)
            return cnt + jnp.sum(msk.astype(jnp.int32))

        cnt = lax.fori_loop(0, CH // 16, scan_body, rem)
        ngr = cnt // G

        def flushg(g, _):
            pltpu.sync_copy(pend.at[pl.ds(_m8(g * G), G)],
                            bin_hbm.at[pl.ds(_m8(tbase + woff + g * G), G)])
            return 0

        lax.fori_loop(0, ngr, flushg, 0)
        off = _m8(ngr * G)
        for k in range(8):
            pend[pl.ds(16 * k, 16)] = pend[pl.ds(off + 16 * k, 16)]
        return (cnt - ngr * G, woff + ngr * G)

    rem, woff = lax.fori_loop(0, NCHUNK, chunk_body, (0, 0))

    kpad = NBUF * G - lax.rem(woff + rem, NBUF * G)
    for k in range(NBUF * G // 16):
        fsrc = 16 * k + iota
        w = lax.bitwise_or(lax.shift_left(fsrc, 9),
                           jnp.full((16,), NLOC, jnp.int32))
        plsc.store_scatter(pend, [rem + fsrc], w, mask=(fsrc < kpad))
    nfin = rem + kpad

    def flushf(g, _):
        pltpu.sync_copy(pend.at[pl.ds(_m8(g * G), G)],
                        bin_hbm.at[pl.ds(_m8(tbase + woff + g * G), G)])
        return 0

    lax.fori_loop(0, nfin // G, flushf, 0)
    total = woff + nfin

    _ring_accum(m_hbm, bin_hbm, tbase, total, pend, gidx, dlv, rows, acc,
                [sem0, sem1, sem2, sem3])
    _write_slab(acc, hn_hbm, wid)
    cntv[:] = jnp.full((16,), total, jnp.int32)
    pltpu.sync_copy(cntv, cnt_hbm.at[pl.ds(_m8(wid * 16), 16)])


def _sc_d2(m_hbm, bin_hbm, cnt_hbm, hn_hbm,
           pend, gidx, dlv, rows, acc, cntv, sem0, sem1, sem2, sem3):
    wid = lax.axis_index("s") * 2 + lax.axis_index("c")
    tbase = wid * CAPT
    _zero_acc(acc)
    pltpu.sync_copy(cnt_hbm.at[pl.ds(_m8(wid * 16), 16)], cntv)
    total = cntv[pl.ds(0, 16)][0]
    _ring_accum(m_hbm, bin_hbm, tbase, total, pend, gidx, dlv, rows, acc,
                [sem0, sem1, sem2, sem3])
    _write_slab(acc, hn_hbm, wid)


def _sc_e(su_hbm, sv_hbm, src_hbm, dst_hbm, out_hbm,
          srcv, dstv, ucol, vcol, ov, sem):
    wid = lax.axis_index("s") * 2 + lax.axis_index("c")
    col = lax.bitwise_and(wid, C - 1)
    half = lax.shift_right_logical(wid, 4)
    ebase = half * (E // 2)
    pltpu.sync_copy(su_hbm.at[pl.ds(_m8(col * NPAD), NPAD)], ucol)
    pltpu.sync_copy(sv_hbm.at[pl.ds(_m8(col * NPAD), NPAD)], vcol)

    def sub(s, _):
        off = ebase + s * SCH
        pltpu.sync_copy(src_hbm.at[pl.ds(_m8(off), SCH)], srcv)
        pltpu.sync_copy(dst_hbm.at[pl.ds(_m8(off), SCH)], dstv)

        def add_body(j, _):
            sv_ = srcv[pl.ds(16 * j, 16)]
            dv_ = dstv[pl.ds(16 * j, 16)]
            u = plsc.load_gather(ucol, [sv_])
            v = plsc.load_gather(vcol, [dv_])
            ov[pl.ds(16 * j, 16)] = u + v
            return 0

        lax.fori_loop(0, SCH // 16, add_body, 0)
        pltpu.sync_copy(ov, out_hbm.at[pl.ds(_m8(col * E + off), SCH)])
        return 0

    lax.fori_loop(0, (E // 2) // SCH, sub, 0)


def kernel(x, edge_index, Wpool1, bpool1, Wself1, bself1, Wneigh1,
           Wpool2, bpool2, Wself2, bself2, Wneigh2, Wp, bp):
    src = edge_index[0]
    dst = edge_index[1]
    xp = jnp.zeros((NPAD, D), jnp.float32).at[:N].set(x)

    f32 = jnp.float32
    tc = functools.partial(pl.pallas_call)

    m1, xs1 = tc(_tc_a, out_shape=[jax.ShapeDtypeStruct((NPAD, D), f32)] * 2)(
        xp, Wpool1, bpool1.reshape(1, D), Wself1, bself1.reshape(1, D))

    d1 = pl.kernel(
        _sc_d1,
        out_type=[
            jax.ShapeDtypeStruct((NPAD * D,), f32),
            jax.ShapeDtypeStruct((TILES * CAPT,), jnp.int32),
            jax.ShapeDtypeStruct((TILES * 16,), jnp.int32),
        ],
        mesh=_MESH,
        compiler_params=pltpu.CompilerParams(needs_layout_passes=False),
        scratch_types=[
            pltpu.VMEM((CH,), jnp.int32),
            pltpu.VMEM((CH,), jnp.int32),
            pltpu.VMEM((PEND,), jnp.int32),
            pltpu.VMEM((NBUF * G,), jnp.int32),
            pltpu.VMEM((NBUF * G,), jnp.int32),
            pltpu.VMEM((NBUF * G, D), f32),
            pltpu.VMEM((NACC * D,), f32),
            pltpu.VMEM((16,), jnp.int32),
            pltpu.SemaphoreType.DMA,
            pltpu.SemaphoreType.DMA,
            pltpu.SemaphoreType.DMA,
            pltpu.SemaphoreType.DMA,
        ],
    )
    hn1f, binned, counts = d1(m1, src, dst)
    hn1 = hn1f.reshape(NPAD, D)

    m2, xs2 = tc(_tc_b, out_shape=[jax.ShapeDtypeStruct((NPAD, D), f32)] * 2)(
        xs1, hn1, Wneigh1, Wpool2, bpool2.reshape(1, D),
        Wself2, bself2.reshape(1, D))

    d2 = pl.kernel(
        _sc_d2,
        out_type=jax.ShapeDtypeStruct((NPAD * D,), f32),
        mesh=_MESH,
        compiler_params=pltpu.CompilerParams(needs_layout_passes=False),
        scratch_types=[
            pltpu.VMEM((PEND,), jnp.int32),
            pltpu.VMEM((NBUF * G,), jnp.int32),
            pltpu.VMEM((NBUF * G,), jnp.int32),
            pltpu.VMEM((NBUF * G, D), f32),
            pltpu.VMEM((NACC * D,), f32),
            pltpu.VMEM((16,), jnp.int32),
            pltpu.SemaphoreType.DMA,
            pltpu.SemaphoreType.DMA,
            pltpu.SemaphoreType.DMA,
            pltpu.SemaphoreType.DMA,
        ],
    )
    hn2 = d2(m2, binned, counts).reshape(NPAD, D)

    suT, svT = tc(_tc_c,
                  out_shape=[jax.ShapeDtypeStruct((C, NPAD), f32)] * 2)(
        xs2, hn2, Wneigh2, Wp[:, :D], Wp[:, D:], bp.reshape(C, 1))

    e = pl.kernel(
        _sc_e,
        out_type=jax.ShapeDtypeStruct((C * E,), f32),
        mesh=_MESH,
        compiler_params=pltpu.CompilerParams(needs_layout_passes=False),
        scratch_types=[
            pltpu.VMEM((SCH,), jnp.int32),
            pltpu.VMEM((SCH,), jnp.int32),
            pltpu.VMEM((NPAD,), f32),
            pltpu.VMEM((NPAD,), f32),
            pltpu.VMEM((SCH,), f32),
            pltpu.SemaphoreType.DMA,
        ],
    )
    scoreT = e(suT.reshape(-1), svT.reshape(-1), src, dst).reshape(C, E)

    tb = 2560
    score = pl.pallas_call(
        _tc_t,
        grid=(E // tb,),
        in_specs=[pl.BlockSpec((C, tb), lambda i: (0, i))],
        out_specs=pl.BlockSpec((tb, C), lambda i: (i, 0)),
        out_shape=jax.ShapeDtypeStruct((E, C), f32),
    )(scoreT)
    return score


# speedup vs baseline: 2.9669x; 1.0025x over previous
import functools

import jax
import jax.numpy as jnp
from jax import lax
from jax.experimental import pallas as pl
from jax.experimental.pallas import tpu as pltpu
from jax.experimental.pallas import tpu_sc as plsc

N = 10000
NPAD = 10016
E = 320000
D = 128
C = 16
EPS = 1e-12

TILES = 32
NLOC = 313
NACC = NLOC + 1
CH = 2000
NCHUNK = E // CH
G = 128
NBUF = 4
PEND = 2304
CAPT = E + NBUF * G
SCH = 2000


def _l2n(h):
    n = jnp.sqrt(jnp.sum(h * h, axis=1, keepdims=True))
    return h / jnp.maximum(n, EPS)


def _tc_a(x_ref, wp_ref, bp_ref, ws_ref, bs_ref, m_ref, xs_ref):
    xb = x_ref[:]
    dn = (((1,), (1,)), ((), ()))
    m_ref[:] = jnp.maximum(
        lax.dot_general(xb, wp_ref[:], dn, preferred_element_type=jnp.float32)
        + bp_ref[:], 0.0)
    xs_ref[:] = (
        lax.dot_general(xb, ws_ref[:], dn, preferred_element_type=jnp.float32)
        + bs_ref[:])


def _tc_b(xs_ref, hn_ref, wn_ref, wp_ref, bp_ref, ws_ref, bs_ref,
          m_ref, xs2_ref):
    dn = (((1,), (1,)), ((), ()))
    h = xs_ref[:] + lax.dot_general(hn_ref[:], wn_ref[:], dn,
                                    preferred_element_type=jnp.float32)
    h = _l2n(jnp.maximum(h, 0.0))
    m_ref[:] = jnp.maximum(
        lax.dot_general(h, wp_ref[:], dn, preferred_element_type=jnp.float32)
        + bp_ref[:], 0.0)
    xs2_ref[:] = (
        lax.dot_general(h, ws_ref[:], dn, preferred_element_type=jnp.float32)
        + bs_ref[:])


def _tc_c(xs_ref, hn_ref, wn_ref, wpu_ref, wpv_ref, bp_ref,
          su_ref, sv_ref):
    dn = (((1,), (1,)), ((), ()))
    h = xs_ref[:] + lax.dot_general(hn_ref[:], wn_ref[:], dn,
                                    preferred_element_type=jnp.float32)
    h = _l2n(h)
    su_ref[:] = (
        lax.dot_general(wpu_ref[:], h, dn, preferred_element_type=jnp.float32)
        + bp_ref[:])
    sv_ref[:] = lax.dot_general(wpv_ref[:], h, dn,
                                preferred_element_type=jnp.float32)


def _tc_t(t_ref, o_ref):
    o_ref[:] = t_ref[:].T


_MESH = plsc.VectorSubcoreMesh(core_axis_name="c", subcore_axis_name="s",
                               num_cores=2, num_subcores=16)


def _m8(v):
    return pl.multiple_of(v, 8)


def _ring_accum(m_hbm, bin_hbm, tbase, total, bw_ref, gidx_ref, dlv_ref,
                rows_ref, acc_ref, sems):
    ngroups = total // G

    def fire(g, b):
        pltpu.sync_copy(bin_hbm.at[pl.ds(_m8(tbase + g * G), G)],
                        bw_ref.at[pl.ds(b * G, G)])
        for k in range(G // 16):
            p = bw_ref[pl.ds(b * G + 16 * k, 16)]
            gidx_ref[pl.ds(b * G + 16 * k, 16)] = \
                lax.shift_right_logical(p, 9)
            dlv_ref[pl.ds(b * G + 16 * k, 16)] = lax.bitwise_and(p, 511)
        pltpu.async_copy(m_hbm.at[gidx_ref.at[pl.ds(b * G, G)]],
                         rows_ref.at[pl.ds(b * G, G)], sems[b])

    def wait(b):
        pltpu.make_async_copy(m_hbm.at[gidx_ref.at[pl.ds(b * G, G)]],
                              rows_ref.at[pl.ds(b * G, G)], sems[b]).wait()

    for b in range(NBUF):
        fire(b, b)

    def step(s, _):
        for b in range(NBUF):
            g = s * NBUF + b
            wait(b)

            def body(q, _, b=b):
                dv = dlv_ref[pl.ds(b * G + 16 * q, 16)]
                for k in range(16):
                    base = dv[k] * D
                    j = b * G + 16 * q + k
                    for cg in range(D // 16):
                        a = acc_ref[pl.ds(base + 16 * cg, 16)]
                        r = rows_ref[j, pl.ds(16 * cg, 16)]
                        acc_ref[pl.ds(base + 16 * cg, 16)] = jnp.maximum(a, r)
                return 0

            lax.fori_loop(0, G // 16, body, 0)
            fire(lax.rem(g + NBUF, ngroups), b)
        return 0

    lax.fori_loop(0, ngroups // NBUF, step, 0)
    for b in range(NBUF):
        wait(b)


def _zero_acc(acc_ref):
    z = jnp.zeros((16,), jnp.float32)

    def zb(i, _):
        for k in range(8):
            acc_ref[pl.ds(i * 128 + 16 * k, 16)] = z
        return 0

    lax.fori_loop(0, NACC, zb, 0)


def _write_slab(acc_ref, hn_hbm, wid):
    pltpu.sync_copy(acc_ref.at[pl.ds(0, NLOC * D)],
                    hn_hbm.at[pl.ds(_m8(wid * NLOC * D), NLOC * D)])


def _sc_d1(m_hbm, src_hbm, dst_hbm, hn_hbm, bin_hbm, cnt_hbm,
           srcv, dstv, pend, gidx, dlv, rows, acc, cntv,
           sem0, sem1, sem2, sem3):
    wid = lax.axis_index("s") * 2 + lax.axis_index("c")
    lo = wid * NLOC
    hi = lo + NLOC
    tbase = wid * CAPT
    _zero_acc(acc)
    iota = lax.iota(jnp.int32, 16)

    def chunk_body(c, carry):
        rem, woff = carry
        pltpu.sync_copy(src_hbm.at[pl.ds(_m8(c * CH), CH)], srcv)
        pltpu.sync_copy(dst_hbm.at[pl.ds(_m8(c * CH), CH)], dstv)

        def scan_body(i, cnt):
            d = dstv[pl.ds(16 * i, 16)]
            s = srcv[pl.ds(16 * i, 16)]
            msk = jnp.logical_and(d >= lo, d < hi)
            p = lax.bitwise_or(lax.shift_left(s, 9),
                               lax.bitwise_and(d - lo, 511))
            cs = plsc.cumsum(msk.astype(jnp.int32))
            idx = cnt + cs - 1
            plsc.store_scatter(pend, [idx], p, mask=msk)
            return cnt + jnp.sum(msk.astype(jnp.int32))

        cnt = lax.fori_loop(0, CH // 16, scan_body, rem)
        ngr = cnt // G

        def flushg(g, _):
            pltpu.sync_copy(pend.at[pl.ds(_m8(g * G), G)],
                            bin_hbm.at[pl.ds(_m8(tbase + woff + g * G), G)])
            return 0

        lax.fori_loop(0, ngr, flushg, 0)
        off = _m8(ngr * G)
        for k in range(8):
            pend[pl.ds(16 * k, 16)] = pend[pl.ds(off + 16 * k, 16)]
        return (cnt - ngr * G, woff + ngr * G)

    rem, woff = lax.fori_loop(0, NCHUNK, chunk_body, (0, 0))

    kpad = NBUF * G - lax.rem(woff + rem, NBUF * G)
    for k in range(NBUF * G // 16):
        fsrc = 16 * k + iota
        w = lax.bitwise_or(lax.shift_left(fsrc, 9),
                           jnp.full((16,), NLOC, jnp.int32))
        plsc.store_scatter(pend, [rem + fsrc], w, mask=(fsrc < kpad))
    nfin = rem + kpad

    def flushf(g, _):
        pltpu.sync_copy(pend.at[pl.ds(_m8(g * G), G)],
                        bin_hbm.at[pl.ds(_m8(tbase + woff + g * G), G)])
        return 0

    lax.fori_loop(0, nfin // G, flushf, 0)
    total = woff + nfin

    _ring_accum(m_hbm, bin_hbm, tbase, total, pend, gidx, dlv, rows, acc,
                [sem0, sem1, sem2, sem3])
    _write_slab(acc, hn_hbm, wid)
    cntv[:] = jnp.full((16,), total, jnp.int32)
    pltpu.sync_copy(cntv, cnt_hbm.at[pl.ds(_m8(wid * 16), 16)])


def _sc_d2(m_hbm, bin_hbm, cnt_hbm, hn_hbm,
           pend, gidx, dlv, rows, acc, cntv, sem0, sem1, sem2, sem3):
    wid = lax.axis_index("s") * 2 + lax.axis_index("c")
    tbase = wid * CAPT
    _zero_acc(acc)
    pltpu.sync_copy(cnt_hbm.at[pl.ds(_m8(wid * 16), 16)], cntv)
    total = cntv[pl.ds(0, 16)][0]
    _ring_accum(m_hbm, bin_hbm, tbase, total, pend, gidx, dlv, rows, acc,
                [sem0, sem1, sem2, sem3])
    _write_slab(acc, hn_hbm, wid)


def _sc_e(su_hbm, sv_hbm, src_hbm, dst_hbm, out_hbm,
          srcv, dstv, ucol, vcol, ov, sem):
    wid = lax.axis_index("s") * 2 + lax.axis_index("c")
    col = lax.bitwise_and(wid, C - 1)
    half = lax.shift_right_logical(wid, 4)
    ebase = half * (E // 2)
    pltpu.sync_copy(su_hbm.at[pl.ds(_m8(col * NPAD), NPAD)], ucol)
    pltpu.sync_copy(sv_hbm.at[pl.ds(_m8(col * NPAD), NPAD)], vcol)

    def sub(s, _):
        off = ebase + s * SCH
        pltpu.sync_copy(src_hbm.at[pl.ds(_m8(off), SCH)], srcv)
        pltpu.sync_copy(dst_hbm.at[pl.ds(_m8(off), SCH)], dstv)

        def add_body(j, _):
            sv_ = srcv[pl.ds(16 * j, 16)]
            dv_ = dstv[pl.ds(16 * j, 16)]
            u = plsc.load_gather(ucol, [sv_])
            v = plsc.load_gather(vcol, [dv_])
            ov[pl.ds(16 * j, 16)] = u + v
            return 0

        lax.fori_loop(0, SCH // 16, add_body, 0)
        pltpu.sync_copy(ov, out_hbm.at[pl.ds(_m8(col * E + off), SCH)])
        return 0

    lax.fori_loop(0, (E // 2) // SCH, sub, 0)


def kernel(x, edge_index, Wpool1, bpool1, Wself1, bself1, Wneigh1,
           Wpool2, bpool2, Wself2, bself2, Wneigh2, Wp, bp):
    src = edge_index[0]
    dst = edge_index[1]
    xp = jnp.zeros((NPAD, D), jnp.float32).at[:N].set(x)

    f32 = jnp.float32
    tc = functools.partial(pl.pallas_call)

    m1, xs1 = tc(_tc_a, out_shape=[jax.ShapeDtypeStruct((NPAD, D), f32)] * 2)(
        xp, Wpool1, bpool1.reshape(1, D), Wself1, bself1.reshape(1, D))

    d1 = pl.kernel(
        _sc_d1,
        out_type=[
            jax.ShapeDtypeStruct((NPAD * D,), f32),
            jax.ShapeDtypeStruct((TILES * CAPT,), jnp.int32),
            jax.ShapeDtypeStruct((TILES * 16,), jnp.int32),
        ],
        mesh=_MESH,
        compiler_params=pltpu.CompilerParams(needs_layout_passes=False),
        scratch_types=[
            pltpu.VMEM((CH,), jnp.int32),
            pltpu.VMEM((CH,), jnp.int32),
            pltpu.VMEM((PEND,), jnp.int32),
            pltpu.VMEM((NBUF * G,), jnp.int32),
            pltpu.VMEM((NBUF * G,), jnp.int32),
            pltpu.VMEM((NBUF * G, D), f32),
            pltpu.VMEM((NACC * D,), f32),
            pltpu.VMEM((16,), jnp.int32),
            pltpu.SemaphoreType.DMA,
            pltpu.SemaphoreType.DMA,
            pltpu.SemaphoreType.DMA,
            pltpu.SemaphoreType.DMA,
        ],
    )
    hn1f, binned, counts = d1(m1, src, dst)
    hn1 = hn1f.reshape(NPAD, D)

    m2, xs2 = tc(_tc_b, out_shape=[jax.ShapeDtypeStruct((NPAD, D), f32)] * 2)(
        xs1, hn1, Wneigh1, Wpool2, bpool2.reshape(1, D),
        Wself2, bself2.reshape(1, D))

    d2 = pl.kernel(
        _sc_d2,
        out_type=jax.ShapeDtypeStruct((NPAD * D,), f32),
        mesh=_MESH,
        compiler_params=pltpu.CompilerParams(needs_layout_passes=False),
        scratch_types=[
            pltpu.VMEM((PEND,), jnp.int32),
            pltpu.VMEM((NBUF * G,), jnp.int32),
            pltpu.VMEM((NBUF * G,), jnp.int32),
            pltpu.VMEM((NBUF * G, D), f32),
            pltpu.VMEM((NACC * D,), f32),
            pltpu.VMEM((16,), jnp.int32),
            pltpu.SemaphoreType.DMA,
            pltpu.SemaphoreType.DMA,
            pltpu.SemaphoreType.DMA,
            pltpu.SemaphoreType.DMA,
        ],
    )
    hn2 = d2(m2, binned, counts).reshape(NPAD, D)

    suT, svT = tc(_tc_c,
                  out_shape=[jax.ShapeDtypeStruct((C, NPAD), f32)] * 2)(
        xs2, hn2, Wneigh2, Wp[:, :D], Wp[:, D:], bp.reshape(C, 1))

    e = pl.kernel(
        _sc_e,
        out_type=jax.ShapeDtypeStruct((C * E,), f32),
        mesh=_MESH,
        compiler_params=pltpu.CompilerParams(needs_layout_passes=False),
        scratch_types=[
            pltpu.VMEM((SCH,), jnp.int32),
            pltpu.VMEM((SCH,), jnp.int32),
            pltpu.VMEM((NPAD,), f32),
            pltpu.VMEM((NPAD,), f32),
            pltpu.VMEM((SCH,), f32),
            pltpu.SemaphoreType.DMA,
        ],
    )
    scoreT = e(suT.reshape(-1), svT.reshape(-1), src, dst).reshape(C, E)

    tb = 2560
    score = pl.pallas_call(
        _tc_t,
        grid=(E // tb,),
        in_specs=[pl.BlockSpec((C, tb), lambda i: (0, i))],
        out_specs=pl.BlockSpec((tb, C), lambda i: (i, 0)),
        out_shape=jax.ShapeDtypeStruct((E, C), f32),
    )(scoreT)
    return score


# speedup vs baseline: 2.9699x; 1.0010x over previous
import functools

import jax
import jax.numpy as jnp
from jax import lax
from jax.experimental import pallas as pl
from jax.experimental.pallas import tpu as pltpu
from jax.experimental.pallas import tpu_sc as plsc

N = 10000
NPAD = 10016
E = 320000
D = 128
C = 16
EPS = 1e-12

TILES = 32
NLOC = 313
NACC = NLOC + 1
CH = 2000
NCHUNK = E // CH
G = 128
NBUF = 2
PEND = 2304
CAPT = E + NBUF * G
SCH = 2000


def _l2n(h):
    n = jnp.sqrt(jnp.sum(h * h, axis=1, keepdims=True))
    return h / jnp.maximum(n, EPS)


def _tc_a(x_ref, wp_ref, bp_ref, ws_ref, bs_ref, m_ref, xs_ref):
    xb = x_ref[:]
    dn = (((1,), (1,)), ((), ()))
    m_ref[:] = jnp.maximum(
        lax.dot_general(xb, wp_ref[:], dn, preferred_element_type=jnp.float32)
        + bp_ref[:], 0.0)
    xs_ref[:] = (
        lax.dot_general(xb, ws_ref[:], dn, preferred_element_type=jnp.float32)
        + bs_ref[:])


def _tc_b(xs_ref, hn_ref, wn_ref, wp_ref, bp_ref, ws_ref, bs_ref,
          m_ref, xs2_ref):
    dn = (((1,), (1,)), ((), ()))
    h = xs_ref[:] + lax.dot_general(hn_ref[:], wn_ref[:], dn,
                                    preferred_element_type=jnp.float32)
    h = _l2n(jnp.maximum(h, 0.0))
    m_ref[:] = jnp.maximum(
        lax.dot_general(h, wp_ref[:], dn, preferred_element_type=jnp.float32)
        + bp_ref[:], 0.0)
    xs2_ref[:] = (
        lax.dot_general(h, ws_ref[:], dn, preferred_element_type=jnp.float32)
        + bs_ref[:])


def _tc_c(xs_ref, hn_ref, wn_ref, wpu_ref, wpv_ref, bp_ref,
          su_ref, sv_ref):
    dn = (((1,), (1,)), ((), ()))
    h = xs_ref[:] + lax.dot_general(hn_ref[:], wn_ref[:], dn,
                                    preferred_element_type=jnp.float32)
    h = _l2n(h)
    su_ref[:] = (
        lax.dot_general(wpu_ref[:], h, dn, preferred_element_type=jnp.float32)
        + bp_ref[:])
    sv_ref[:] = lax.dot_general(wpv_ref[:], h, dn,
                                preferred_element_type=jnp.float32)


def _tc_t(t_ref, o_ref):
    o_ref[:] = t_ref[:].T


_MESH = plsc.VectorSubcoreMesh(core_axis_name="c", subcore_axis_name="s",
                               num_cores=2, num_subcores=16)


def _m8(v):
    return pl.multiple_of(v, 8)


def _ring_accum(m_hbm, bin_hbm, tbase, total, bw_ref, gidx_ref, dlv_ref,
                rows_ref, acc_ref, acc2_ref, sems):
    ngroups = total // G

    def fire(g, b):
        pltpu.sync_copy(bin_hbm.at[pl.ds(_m8(tbase + g * G), G)],
                        bw_ref.at[pl.ds(b * G, G)])
        for k in range(G // 16):
            p = bw_ref[pl.ds(b * G + 16 * k, 16)]
            gidx_ref[pl.ds(b * G + 16 * k, 16)] = \
                lax.shift_right_logical(p, 9)
            dlv_ref[pl.ds(b * G + 16 * k, 16)] = lax.bitwise_and(p, 511)
        pltpu.async_copy(m_hbm.at[gidx_ref.at[pl.ds(b * G, G)]],
                         rows_ref.at[pl.ds(b * G, G)], sems[b])

    def wait(b):
        pltpu.make_async_copy(m_hbm.at[gidx_ref.at[pl.ds(b * G, G)]],
                              rows_ref.at[pl.ds(b * G, G)], sems[b]).wait()

    for b in range(NBUF):
        fire(b, b)

    def step(s, _):
        for b in range(NBUF):
            g = s * NBUF + b
            wait(b)

            def body(q, _, b=b):
                dv = dlv_ref[pl.ds(b * G + 16 * q, 16)]
                for k in range(16):
                    base = dv[k] * D
                    j = b * G + 16 * q + k
                    ar = acc_ref if k % 2 == 0 else acc2_ref
                    for cg in range(D // 16):
                        a = ar[pl.ds(base + 16 * cg, 16)]
                        r = rows_ref[j, pl.ds(16 * cg, 16)]
                        ar[pl.ds(base + 16 * cg, 16)] = jnp.maximum(a, r)
                return 0

            lax.fori_loop(0, G // 16, body, 0)
            fire(lax.rem(g + NBUF, ngroups), b)
        return 0

    lax.fori_loop(0, ngroups // NBUF, step, 0)
    for b in range(NBUF):
        wait(b)


def _zero_acc(acc_ref):
    z = jnp.zeros((16,), jnp.float32)

    def zb(i, _):
        for k in range(8):
            acc_ref[pl.ds(i * 128 + 16 * k, 16)] = z
        return 0

    lax.fori_loop(0, NACC, zb, 0)


def _write_slab(acc_ref, acc2_ref, hn_hbm, wid):
    def mb(i, _):
        for k in range(8):
            o = i * 128 + 16 * k
            acc_ref[pl.ds(o, 16)] = jnp.maximum(acc_ref[pl.ds(o, 16)],
                                                acc2_ref[pl.ds(o, 16)])
        return 0

    lax.fori_loop(0, NLOC, mb, 0)
    pltpu.sync_copy(acc_ref.at[pl.ds(0, NLOC * D)],
                    hn_hbm.at[pl.ds(_m8(wid * NLOC * D), NLOC * D)])


def _sc_d1(m_hbm, src_hbm, dst_hbm, hn_hbm, bin_hbm, cnt_hbm,
           srcv, dstv, pend, gidx, dlv, rows, acc, acc2, cntv,
           sem0, sem1):
    wid = lax.axis_index("s") * 2 + lax.axis_index("c")
    lo = wid * NLOC
    hi = lo + NLOC
    tbase = wid * CAPT
    _zero_acc(acc)
    _zero_acc(acc2)
    iota = lax.iota(jnp.int32, 16)

    def chunk_body(c, carry):
        rem, woff = carry
        pltpu.sync_copy(src_hbm.at[pl.ds(_m8(c * CH), CH)], srcv)
        pltpu.sync_copy(dst_hbm.at[pl.ds(_m8(c * CH), CH)], dstv)

        def scan_body(i, cnt):
            d = dstv[pl.ds(16 * i, 16)]
            s = srcv[pl.ds(16 * i, 16)]
            msk = jnp.logical_and(d >= lo, d < hi)
            p = lax.bitwise_or(lax.shift_left(s, 9),
                               lax.bitwise_and(d - lo, 511))
            cs = plsc.cumsum(msk.astype(jnp.int32))
            idx = cnt + cs - 1
            plsc.store_scatter(pend, [idx], p, mask=msk)
            return cnt + jnp.sum(msk.astype(jnp.int32))

        cnt = lax.fori_loop(0, CH // 16, scan_body, rem)
        ngr = cnt // G

        def flushg(g, _):
            pltpu.sync_copy(pend.at[pl.ds(_m8(g * G), G)],
                            bin_hbm.at[pl.ds(_m8(tbase + woff + g * G), G)])
            return 0

        lax.fori_loop(0, ngr, flushg, 0)
        off = _m8(ngr * G)
        for k in range(8):
            pend[pl.ds(16 * k, 16)] = pend[pl.ds(off + 16 * k, 16)]
        return (cnt - ngr * G, woff + ngr * G)

    rem, woff = lax.fori_loop(0, NCHUNK, chunk_body, (0, 0))

    kpad = NBUF * G - lax.rem(woff + rem, NBUF * G)
    for k in range(NBUF * G // 16):
        fsrc = 16 * k + iota
        w = lax.bitwise_or(lax.shift_left(fsrc, 9),
                           jnp.full((16,), NLOC, jnp.int32))
        plsc.store_scatter(pend, [rem + fsrc], w, mask=(fsrc < kpad))
    nfin = rem + kpad

    def flushf(g, _):
        pltpu.sync_copy(pend.at[pl.ds(_m8(g * G), G)],
                        bin_hbm.at[pl.ds(_m8(tbase + woff + g * G), G)])
        return 0

    lax.fori_loop(0, nfin // G, flushf, 0)
    total = woff + nfin

    _ring_accum(m_hbm, bin_hbm, tbase, total, pend, gidx, dlv, rows, acc,
                acc2, [sem0, sem1])
    _write_slab(acc, acc2, hn_hbm, wid)
    cntv[:] = jnp.full((16,), total, jnp.int32)
    pltpu.sync_copy(cntv, cnt_hbm.at[pl.ds(_m8(wid * 16), 16)])


def _sc_d2(m_hbm, bin_hbm, cnt_hbm, hn_hbm,
           pend, gidx, dlv, rows, acc, acc2, cntv, sem0, sem1):
    wid = lax.axis_index("s") * 2 + lax.axis_index("c")
    tbase = wid * CAPT
    _zero_acc(acc)
    _zero_acc(acc2)
    pltpu.sync_copy(cnt_hbm.at[pl.ds(_m8(wid * 16), 16)], cntv)
    total = cntv[pl.ds(0, 16)][0]
    _ring_accum(m_hbm, bin_hbm, tbase, total, pend, gidx, dlv, rows, acc,
                acc2, [sem0, sem1])
    _write_slab(acc, acc2, hn_hbm, wid)


def _sc_e(su_hbm, sv_hbm, src_hbm, dst_hbm, out_hbm,
          srcv, dstv, ucol, vcol, ov, sem):
    wid = lax.axis_index("s") * 2 + lax.axis_index("c")
    col = lax.bitwise_and(wid, C - 1)
    half = lax.shift_right_logical(wid, 4)
    ebase = half * (E // 2)
    pltpu.sync_copy(su_hbm.at[pl.ds(_m8(col * NPAD), NPAD)], ucol)
    pltpu.sync_copy(sv_hbm.at[pl.ds(_m8(col * NPAD), NPAD)], vcol)

    def sub(s, _):
        off = ebase + s * SCH
        pltpu.sync_copy(src_hbm.at[pl.ds(_m8(off), SCH)], srcv)
        pltpu.sync_copy(dst_hbm.at[pl.ds(_m8(off), SCH)], dstv)

        def add_body(j, _):
            sv_ = srcv[pl.ds(16 * j, 16)]
            dv_ = dstv[pl.ds(16 * j, 16)]
            u = plsc.load_gather(ucol, [sv_])
            v = plsc.load_gather(vcol, [dv_])
            ov[pl.ds(16 * j, 16)] = u + v
            return 0

        lax.fori_loop(0, SCH // 16, add_body, 0)
        pltpu.sync_copy(ov, out_hbm.at[pl.ds(_m8(col * E + off), SCH)])
        return 0

    lax.fori_loop(0, (E // 2) // SCH, sub, 0)


def kernel(x, edge_index, Wpool1, bpool1, Wself1, bself1, Wneigh1,
           Wpool2, bpool2, Wself2, bself2, Wneigh2, Wp, bp):
    src = edge_index[0]
    dst = edge_index[1]
    xp = jnp.zeros((NPAD, D), jnp.float32).at[:N].set(x)

    f32 = jnp.float32
    tc = functools.partial(pl.pallas_call)

    m1, xs1 = tc(_tc_a, out_shape=[jax.ShapeDtypeStruct((NPAD, D), f32)] * 2)(
        xp, Wpool1, bpool1.reshape(1, D), Wself1, bself1.reshape(1, D))

    d1 = pl.kernel(
        _sc_d1,
        out_type=[
            jax.ShapeDtypeStruct((NPAD * D,), f32),
            jax.ShapeDtypeStruct((TILES * CAPT,), jnp.int32),
            jax.ShapeDtypeStruct((TILES * 16,), jnp.int32),
        ],
        mesh=_MESH,
        compiler_params=pltpu.CompilerParams(needs_layout_passes=False),
        scratch_types=[
            pltpu.VMEM((CH,), jnp.int32),
            pltpu.VMEM((CH,), jnp.int32),
            pltpu.VMEM((PEND,), jnp.int32),
            pltpu.VMEM((NBUF * G,), jnp.int32),
            pltpu.VMEM((NBUF * G,), jnp.int32),
            pltpu.VMEM((NBUF * G, D), f32),
            pltpu.VMEM((NACC * D,), f32),
            pltpu.VMEM((NACC * D,), f32),
            pltpu.VMEM((16,), jnp.int32),
            pltpu.SemaphoreType.DMA,
            pltpu.SemaphoreType.DMA,
        ],
    )
    hn1f, binned, counts = d1(m1, src, dst)
    hn1 = hn1f.reshape(NPAD, D)

    m2, xs2 = tc(_tc_b, out_shape=[jax.ShapeDtypeStruct((NPAD, D), f32)] * 2)(
        xs1, hn1, Wneigh1, Wpool2, bpool2.reshape(1, D),
        Wself2, bself2.reshape(1, D))

    d2 = pl.kernel(
        _sc_d2,
        out_type=jax.ShapeDtypeStruct((NPAD * D,), f32),
        mesh=_MESH,
        compiler_params=pltpu.CompilerParams(needs_layout_passes=False),
        scratch_types=[
            pltpu.VMEM((PEND,), jnp.int32),
            pltpu.VMEM((NBUF * G,), jnp.int32),
            pltpu.VMEM((NBUF * G,), jnp.int32),
            pltpu.VMEM((NBUF * G, D), f32),
            pltpu.VMEM((NACC * D,), f32),
            pltpu.VMEM((NACC * D,), f32),
            pltpu.VMEM((16,), jnp.int32),
            pltpu.SemaphoreType.DMA,
            pltpu.SemaphoreType.DMA,
        ],
    )
    hn2 = d2(m2, binned, counts).reshape(NPAD, D)

    suT, svT = tc(_tc_c,
                  out_shape=[jax.ShapeDtypeStruct((C, NPAD), f32)] * 2)(
        xs2, hn2, Wneigh2, Wp[:, :D], Wp[:, D:], bp.reshape(C, 1))

    e = pl.kernel(
        _sc_e,
        out_type=jax.ShapeDtypeStruct((C * E,), f32),
        mesh=_MESH,
        compiler_params=pltpu.CompilerParams(needs_layout_passes=False),
        scratch_types=[
            pltpu.VMEM((SCH,), jnp.int32),
            pltpu.VMEM((SCH,), jnp.int32),
            pltpu.VMEM((NPAD,), f32),
            pltpu.VMEM((NPAD,), f32),
            pltpu.VMEM((SCH,), f32),
            pltpu.SemaphoreType.DMA,
        ],
    )
    scoreT = e(suT.reshape(-1), svT.reshape(-1), src, dst).reshape(C, E)

    tb = 2560
    score = pl.pallas_call(
        _tc_t,
        grid=(E // tb,),
        in_specs=[pl.BlockSpec((C, tb), lambda i: (0, i))],
        out_specs=pl.BlockSpec((tb, C), lambda i: (i, 0)),
        out_shape=jax.ShapeDtypeStruct((E, C), f32),
    )(scoreT)
    return score


# speedup vs baseline: 3.0400x; 1.0236x over previous
import functools

import jax
import jax.numpy as jnp
from jax import lax
from jax.experimental import pallas as pl
from jax.experimental.pallas import tpu as pltpu
from jax.experimental.pallas import tpu_sc as plsc

N = 10000
NPAD = 10016
E = 320000
D = 128
C = 16
EPS = 1e-12

TILES = 32
NLOC = 313
NACC = NLOC + 1
CH = 2000
NCHUNK = E // CH
G = 128
NBUF = 2
PEND = 2304
BGR = 16
BW = BGR * G
CAPT = E + NBUF * G
SCH = 2000


def _l2n(h):
    n = jnp.sqrt(jnp.sum(h * h, axis=1, keepdims=True))
    return h / jnp.maximum(n, EPS)


def _tc_a(x_ref, wp_ref, bp_ref, ws_ref, bs_ref, m_ref, xs_ref):
    xb = x_ref[:]
    dn = (((1,), (1,)), ((), ()))
    m_ref[:] = jnp.maximum(
        lax.dot_general(xb, wp_ref[:], dn, preferred_element_type=jnp.float32)
        + bp_ref[:], 0.0)
    xs_ref[:] = (
        lax.dot_general(xb, ws_ref[:], dn, preferred_element_type=jnp.float32)
        + bs_ref[:])


def _tc_b(xs_ref, hn_ref, wn_ref, wp_ref, bp_ref, ws_ref, bs_ref,
          m_ref, xs2_ref):
    dn = (((1,), (1,)), ((), ()))
    h = xs_ref[:] + lax.dot_general(hn_ref[:], wn_ref[:], dn,
                                    preferred_element_type=jnp.float32)
    h = _l2n(jnp.maximum(h, 0.0))
    m_ref[:] = jnp.maximum(
        lax.dot_general(h, wp_ref[:], dn, preferred_element_type=jnp.float32)
        + bp_ref[:], 0.0)
    xs2_ref[:] = (
        lax.dot_general(h, ws_ref[:], dn, preferred_element_type=jnp.float32)
        + bs_ref[:])


def _tc_c(xs_ref, hn_ref, wn_ref, wpu_ref, wpv_ref, bp_ref,
          su_ref, sv_ref):
    dn = (((1,), (1,)), ((), ()))
    h = xs_ref[:] + lax.dot_general(hn_ref[:], wn_ref[:], dn,
                                    preferred_element_type=jnp.float32)
    h = _l2n(h)
    su_ref[:] = (
        lax.dot_general(wpu_ref[:], h, dn, preferred_element_type=jnp.float32)
        + bp_ref[:])
    sv_ref[:] = lax.dot_general(wpv_ref[:], h, dn,
                                preferred_element_type=jnp.float32)


def _tc_t(t_ref, o_ref):
    o_ref[:] = t_ref[:].T


_MESH = plsc.VectorSubcoreMesh(core_axis_name="c", subcore_axis_name="s",
                               num_cores=2, num_subcores=16)


def _m8(v):
    return pl.multiple_of(v, 8)


def _ring_accum(m_hbm, bin_hbm, tbase, total, bw_ref, gidx_ref, dlv_ref,
                rows_ref, acc_ref, acc2_ref, sems):
    ngroups = total // G

    def fire(g, b):
        for k in range(G // 16):
            p = bw_ref[pl.ds(g * G + 16 * k, 16)]
            gidx_ref[pl.ds(b * G + 16 * k, 16)] = \
                lax.shift_right_logical(p, 9)
            dlv_ref[pl.ds(b * G + 16 * k, 16)] = lax.bitwise_and(p, 511)
        pltpu.async_copy(m_hbm.at[gidx_ref.at[pl.ds(b * G, G)]],
                         rows_ref.at[pl.ds(b * G, G)], sems[b])

    def wait(b):
        pltpu.make_async_copy(m_hbm.at[gidx_ref.at[pl.ds(b * G, G)]],
                              rows_ref.at[pl.ds(b * G, G)], sems[b]).wait()

    def accum(b):
        def body(q, _):
            dv = dlv_ref[pl.ds(b * G + 16 * q, 16)]
            for k in range(16):
                base = dv[k] * D
                j = b * G + 16 * q + k
                ar = acc_ref if k % 2 == 0 else acc2_ref
                for cg in range(D // 16):
                    a = ar[pl.ds(base + 16 * cg, 16)]
                    r = rows_ref[j, pl.ds(16 * cg, 16)]
                    ar[pl.ds(base + 16 * cg, 16)] = jnp.maximum(a, r)
            return 0

        lax.fori_loop(0, G // 16, body, 0)

    nwin = (ngroups + BGR - 1) // BGR

    def win(u, _):
        pltpu.sync_copy(bin_hbm.at[pl.ds(_m8(tbase + u * BW), BW)],
                        bw_ref.at[pl.ds(0, BW)])
        ng = lax.min(BGR, ngroups - u * BGR)
        for b in range(NBUF):
            fire(b, b)

        def pair(t, _):
            for b in range(NBUF):
                g = NBUF * t + b
                wait(b)
                accum(b)
                fire(lax.min(g + NBUF, ng - NBUF + b), b)
            return 0

        lax.fori_loop(0, ng // NBUF, pair, 0)
        for b in range(NBUF):
            wait(b)
        return 0

    lax.fori_loop(0, nwin, win, 0)


def _zero_acc(acc_ref):
    z = jnp.zeros((16,), jnp.float32)

    def zb(i, _):
        for k in range(8):
            acc_ref[pl.ds(i * 128 + 16 * k, 16)] = z
        return 0

    lax.fori_loop(0, NACC, zb, 0)


def _write_slab(acc_ref, acc2_ref, hn_hbm, wid):
    def mb(i, _):
        for k in range(8):
            o = i * 128 + 16 * k
            acc_ref[pl.ds(o, 16)] = jnp.maximum(acc_ref[pl.ds(o, 16)],
                                                acc2_ref[pl.ds(o, 16)])
        return 0

    lax.fori_loop(0, NLOC, mb, 0)
    pltpu.sync_copy(acc_ref.at[pl.ds(0, NLOC * D)],
                    hn_hbm.at[pl.ds(_m8(wid * NLOC * D), NLOC * D)])


def _sc_d1(m_hbm, src_hbm, dst_hbm, hn_hbm, bin_hbm, cnt_hbm,
           srcv, dstv, pend, gidx, dlv, rows, acc, acc2, cntv,
           sem0, sem1):
    wid = lax.axis_index("s") * 2 + lax.axis_index("c")
    lo = wid * NLOC
    hi = lo + NLOC
    tbase = wid * CAPT
    _zero_acc(acc)
    _zero_acc(acc2)
    iota = lax.iota(jnp.int32, 16)

    def chunk_body(c, carry):
        rem, woff = carry
        pltpu.sync_copy(src_hbm.at[pl.ds(_m8(c * CH), CH)], srcv)
        pltpu.sync_copy(dst_hbm.at[pl.ds(_m8(c * CH), CH)], dstv)

        def scan_body(i, cnt):
            d = dstv[pl.ds(16 * i, 16)]
            s = srcv[pl.ds(16 * i, 16)]
            msk = jnp.logical_and(d >= lo, d < hi)
            p = lax.bitwise_or(lax.shift_left(s, 9),
                               lax.bitwise_and(d - lo, 511))
            cs = plsc.cumsum(msk.astype(jnp.int32))
            idx = cnt + cs - 1
            plsc.store_scatter(pend, [idx], p, mask=msk)
            return cnt + jnp.sum(msk.astype(jnp.int32))

        cnt = lax.fori_loop(0, CH // 16, scan_body, rem)
        ngr = cnt // G

        def flushg(g, _):
            pltpu.sync_copy(pend.at[pl.ds(_m8(g * G), G)],
                            bin_hbm.at[pl.ds(_m8(tbase + woff + g * G), G)])
            return 0

        lax.fori_loop(0, ngr, flushg, 0)
        off = _m8(ngr * G)
        for k in range(8):
            pend[pl.ds(16 * k, 16)] = pend[pl.ds(off + 16 * k, 16)]
        return (cnt - ngr * G, woff + ngr * G)

    rem, woff = lax.fori_loop(0, NCHUNK, chunk_body, (0, 0))

    kpad = NBUF * G - lax.rem(woff + rem, NBUF * G)
    for k in range(NBUF * G // 16):
        fsrc = 16 * k + iota
        w = lax.bitwise_or(lax.shift_left(fsrc, 9),
                           jnp.full((16,), NLOC, jnp.int32))
        plsc.store_scatter(pend, [rem + fsrc], w, mask=(fsrc < kpad))
    nfin = rem + kpad

    def flushf(g, _):
        pltpu.sync_copy(pend.at[pl.ds(_m8(g * G), G)],
                        bin_hbm.at[pl.ds(_m8(tbase + woff + g * G), G)])
        return 0

    lax.fori_loop(0, nfin // G, flushf, 0)
    total = woff + nfin

    _ring_accum(m_hbm, bin_hbm, tbase, total, pend, gidx, dlv, rows, acc,
                acc2, [sem0, sem1])
    _write_slab(acc, acc2, hn_hbm, wid)
    cntv[:] = jnp.full((16,), total, jnp.int32)
    pltpu.sync_copy(cntv, cnt_hbm.at[pl.ds(_m8(wid * 16), 16)])


def _sc_d2(m_hbm, bin_hbm, cnt_hbm, hn_hbm,
           pend, gidx, dlv, rows, acc, acc2, cntv, sem0, sem1):
    wid = lax.axis_index("s") * 2 + lax.axis_index("c")
    tbase = wid * CAPT
    _zero_acc(acc)
    _zero_acc(acc2)
    pltpu.sync_copy(cnt_hbm.at[pl.ds(_m8(wid * 16), 16)], cntv)
    total = cntv[pl.ds(0, 16)][0]
    _ring_accum(m_hbm, bin_hbm, tbase, total, pend, gidx, dlv, rows, acc,
                acc2, [sem0, sem1])
    _write_slab(acc, acc2, hn_hbm, wid)


def _sc_e(su_hbm, sv_hbm, src_hbm, dst_hbm, out_hbm,
          srcv, dstv, ucol, vcol, ov, sem):
    wid = lax.axis_index("s") * 2 + lax.axis_index("c")
    col = lax.bitwise_and(wid, C - 1)
    half = lax.shift_right_logical(wid, 4)
    ebase = half * (E // 2)
    pltpu.sync_copy(su_hbm.at[pl.ds(_m8(col * NPAD), NPAD)], ucol)
    pltpu.sync_copy(sv_hbm.at[pl.ds(_m8(col * NPAD), NPAD)], vcol)

    def sub(s, _):
        off = ebase + s * SCH
        pltpu.sync_copy(src_hbm.at[pl.ds(_m8(off), SCH)], srcv)
        pltpu.sync_copy(dst_hbm.at[pl.ds(_m8(off), SCH)], dstv)

        def add_body(j, _):
            sv_ = srcv[pl.ds(16 * j, 16)]
            dv_ = dstv[pl.ds(16 * j, 16)]
            u = plsc.load_gather(ucol, [sv_])
            v = plsc.load_gather(vcol, [dv_])
            ov[pl.ds(16 * j, 16)] = u + v
            return 0

        lax.fori_loop(0, SCH // 16, add_body, 0)
        pltpu.sync_copy(ov, out_hbm.at[pl.ds(_m8(col * E + off), SCH)])
        return 0

    lax.fori_loop(0, (E // 2) // SCH, sub, 0)


def kernel(x, edge_index, Wpool1, bpool1, Wself1, bself1, Wneigh1,
           Wpool2, bpool2, Wself2, bself2, Wneigh2, Wp, bp):
    src = edge_index[0]
    dst = edge_index[1]
    xp = jnp.zeros((NPAD, D), jnp.float32).at[:N].set(x)

    f32 = jnp.float32
    tc = functools.partial(pl.pallas_call)

    m1, xs1 = tc(_tc_a, out_shape=[jax.ShapeDtypeStruct((NPAD, D), f32)] * 2)(
        xp, Wpool1, bpool1.reshape(1, D), Wself1, bself1.reshape(1, D))

    d1 = pl.kernel(
        _sc_d1,
        out_type=[
            jax.ShapeDtypeStruct((NPAD * D,), f32),
            jax.ShapeDtypeStruct((TILES * CAPT + BW,), jnp.int32),
            jax.ShapeDtypeStruct((TILES * 16,), jnp.int32),
        ],
        mesh=_MESH,
        compiler_params=pltpu.CompilerParams(needs_layout_passes=False),
        scratch_types=[
            pltpu.VMEM((CH,), jnp.int32),
            pltpu.VMEM((CH,), jnp.int32),
            pltpu.VMEM((PEND,), jnp.int32),
            pltpu.VMEM((NBUF * G,), jnp.int32),
            pltpu.VMEM((NBUF * G,), jnp.int32),
            pltpu.VMEM((NBUF * G, D), f32),
            pltpu.VMEM((NACC * D,), f32),
            pltpu.VMEM((NACC * D,), f32),
            pltpu.VMEM((16,), jnp.int32),
            pltpu.SemaphoreType.DMA,
            pltpu.SemaphoreType.DMA,
        ],
    )
    hn1f, binned, counts = d1(m1, src, dst)
    hn1 = hn1f.reshape(NPAD, D)

    m2, xs2 = tc(_tc_b, out_shape=[jax.ShapeDtypeStruct((NPAD, D), f32)] * 2)(
        xs1, hn1, Wneigh1, Wpool2, bpool2.reshape(1, D),
        Wself2, bself2.reshape(1, D))

    d2 = pl.kernel(
        _sc_d2,
        out_type=jax.ShapeDtypeStruct((NPAD * D,), f32),
        mesh=_MESH,
        compiler_params=pltpu.CompilerParams(needs_layout_passes=False),
        scratch_types=[
            pltpu.VMEM((PEND,), jnp.int32),
            pltpu.VMEM((NBUF * G,), jnp.int32),
            pltpu.VMEM((NBUF * G,), jnp.int32),
            pltpu.VMEM((NBUF * G, D), f32),
            pltpu.VMEM((NACC * D,), f32),
            pltpu.VMEM((NACC * D,), f32),
            pltpu.VMEM((16,), jnp.int32),
            pltpu.SemaphoreType.DMA,
            pltpu.SemaphoreType.DMA,
        ],
    )
    hn2 = d2(m2, binned, counts).reshape(NPAD, D)

    suT, svT = tc(_tc_c,
                  out_shape=[jax.ShapeDtypeStruct((C, NPAD), f32)] * 2)(
        xs2, hn2, Wneigh2, Wp[:, :D], Wp[:, D:], bp.reshape(C, 1))

    e = pl.kernel(
        _sc_e,
        out_type=jax.ShapeDtypeStruct((C * E,), f32),
        mesh=_MESH,
        compiler_params=pltpu.CompilerParams(needs_layout_passes=False),
        scratch_types=[
            pltpu.VMEM((SCH,), jnp.int32),
            pltpu.VMEM((SCH,), jnp.int32),
            pltpu.VMEM((NPAD,), f32),
            pltpu.VMEM((NPAD,), f32),
            pltpu.VMEM((SCH,), f32),
            pltpu.SemaphoreType.DMA,
        ],
    )
    scoreT = e(suT.reshape(-1), svT.reshape(-1), src, dst).reshape(C, E)

    tb = 2560
    score = pl.pallas_call(
        _tc_t,
        grid=(E // tb,),
        in_specs=[pl.BlockSpec((C, tb), lambda i: (0, i))],
        out_specs=pl.BlockSpec((tb, C), lambda i: (i, 0)),
        out_shape=jax.ShapeDtypeStruct((E, C), f32),
    )(scoreT)
    return score
